# Initial kernel scaffold; baseline (speedup 1.0000x reference)
import jax
import jax.numpy as jnp
from jax.experimental import pallas as pl


def kernel(ft_user, ft_item, bn_g_u, bn_b_u, bn_g_i, bn_b_i, Wq_ui, bq_ui, Wk_ui, Wv_ui, attn_ui, emb_cnt, Wq_ii, bq_ii, Wk_ii, Wv_ii, attn_ii, W_agg, b_agg, W_self, src_ui, dst_ui, src_ii, dst_ii, cnt_ui):
    raise NotImplementedError("write your pallas kernel here")


# speedup vs baseline: 4.1098x; 4.1098x over previous
import functools

import jax
import jax.numpy as jnp
from jax import lax
from jax.experimental import pallas as pl
from jax.experimental.pallas import tpu as pltpu
from jax.experimental.pallas import tpu_sc as plsc

N_U = 10000
N_I = 10000
E1 = 160000
E2 = 160000
D = 128

L = 16
NCORES = 2
NSUB = 16
NW = NCORES * NSUB
CH = 128
CPW = 40
EP = NW * CPW * CH
NSEG = 10240
SEG_T = NSEG // NSUB
NEG = -1e30

_f32 = jnp.float32
_i32 = jnp.int32


def _mesh():
    return plsc.VectorSubcoreMesh(
        core_axis_name="c", subcore_axis_name="s",
        num_cores=NCORES, num_subcores=NSUB)


def _tc1_body(ftu, fti, gu, bu, gi, bi,
              wq1, bq1, wk1, wv1, wq2, bq2, wk2, wv2,
              q1o, k1o, v1o, q2o, k2o, v2o, ftio):
    xu = ftu[...]
    mu = jnp.mean(xu, axis=0, keepdims=True)
    vu = jnp.mean((xu - mu) ** 2, axis=0, keepdims=True)
    xu = (xu - mu) / jnp.sqrt(vu + 1e-5) * gu[...] + bu[...]
    xi = fti[...]
    mi = jnp.mean(xi, axis=0, keepdims=True)
    vi = jnp.mean((xi - mi) ** 2, axis=0, keepdims=True)
    xi = (xi - mi) / jnp.sqrt(vi + 1e-5) * gi[...] + bi[...]
    ftio[...] = xi
    dot = functools.partial(jnp.dot, preferred_element_type=_f32)
    q1o[...] = dot(xu, wq1[...]) + bq1[...]
    k1o[...] = dot(xi, wk1[...])
    v1o[...] = dot(xu, wv1[...])
    q2o[...] = dot(xi, wq2[...]) + bq2[...]
    k2o[...] = dot(xi, wk2[...])
    v2o[...] = dot(xi, wv2[...])


def _tc2_body(aggp, denp, fti, wagg, bagg, wself, out):
    agg = aggp[0, :N_I, :] + aggp[1, :N_I, :]
    den = jnp.sum(denp[:, :N_I], axis=0)
    den = jnp.where(den > 0.0, den, 1.0)
    a = agg / den[:, None]
    dot = functools.partial(jnp.dot, preferred_element_type=_f32)
    out[...] = jnp.maximum(
        dot(a, wagg[...]) + dot(fti[...], wself[...]) + bagg[...], 0.0)


def _s1_body(q1, k1, emb, attn1, q2, k2, attn2,
             src1, dst1, cnt1, src2, dst2,
             s1_out, s2_out,
             idx_a, idx_b, idx_c, qr, kr, cr, attn_v, sc_v,
             sem_a, sem_b, sem_c):
    wid = lax.axis_index("s") * NCORES + lax.axis_index("c")

    def do_etype(qtab, ktab, attn_hbm, src, dst, cnt, out, has_cnt):
        pltpu.sync_copy(attn_hbm, attn_v)

        def chunk_body(c, carry):
            base = (wid * CPW + c) * CH
            pltpu.sync_copy(src.at[pl.ds(base, CH)], idx_a)
            pltpu.sync_copy(dst.at[pl.ds(base, CH)], idx_b)
            cq = pltpu.async_copy(qtab.at[idx_a], qr, sem_a)
            ck = pltpu.async_copy(ktab.at[idx_b], kr, sem_b)
            if has_cnt:
                pltpu.sync_copy(cnt.at[pl.ds(base, CH)], idx_c)
                cc = pltpu.async_copy(emb.at[idx_c], cr, sem_c)
            cq.wait()
            ck.wait()
            if has_cnt:
                cc.wait()

            def grp(g, carry2):
                def edge(r2, vec):
                    r = g * L + r2
                    acc = jnp.zeros((L,), _f32)
                    for j in range(8):
                        sl = pl.ds(j * L, L)
                        t = qr[r, sl] + kr[r, sl]
                        if has_cnt:
                            t = t + cr[r, sl]
                        sig = 1.0 / (1.0 + jnp.exp(-t))
                        acc = acc + attn_v[sl] * sig
                    s = jnp.sum(acc)
                    lane = lax.broadcasted_iota(_i32, (L,), 0)
                    return jnp.where(lane == r2, s, vec)

                vec = lax.fori_loop(0, L, edge, jnp.zeros((L,), _f32))
                sc_v[pl.ds(g * L, L)] = vec
                return carry2

            lax.fori_loop(0, CH // L, grp, 0)
            pltpu.sync_copy(sc_v, out.at[pl.ds(base, CH)])
            return carry

        lax.fori_loop(0, CPW, chunk_body, 0)

    do_etype(q1, k1, attn1, src1, dst1, cnt1, s1_out, True)
    do_etype(q2, k2, attn2, src2, dst2, None, s2_out, False)


def _s2_body(s1, s2, dst1, dst2, m_parts,
             m_priv, idx_v, s_v, a_v, t_v, stage):
    cid = lax.axis_index("c")
    sid = lax.axis_index("s")
    wid = sid * NCORES + cid

    def ini(i, carry):
        m_priv[pl.ds(i * L, L)] = jnp.full((L,), NEG, _f32)
        return carry

    lax.fori_loop(0, NSEG // L, ini, 0)

    def do(scores, dst):
        def chunk(c, carry):
            base = (wid * CPW + c) * CH
            pltpu.sync_copy(dst.at[pl.ds(base, CH)], idx_v)
            pltpu.sync_copy(scores.at[pl.ds(base, CH)], s_v)

            def grp(g, carry2):
                sl = pl.ds(g * L, L)
                idxv = idx_v[sl]
                sv = s_v[sl]

                def cond(st):
                    cur = plsc.load_gather(m_priv, [idxv])
                    return jnp.logical_and(st < L, jnp.any(cur < sv))

                def body(st):
                    cur = plsc.load_gather(m_priv, [idxv])
                    msk = cur < sv
                    plsc.store_scatter(m_priv, [idxv],
                                       jnp.maximum(cur, sv), mask=msk)
                    return st + 1

                lax.while_loop(cond, body, 0)
                return carry2

            lax.fori_loop(0, CH // L, grp, 0)
            return carry

        lax.fori_loop(0, CPW, chunk, 0)

    do(s1, dst1)
    do(s2, dst2)

    pltpu.sync_copy(m_priv, stage.at[sid])
    plsc.subcore_barrier()
    pltpu.sync_copy(stage.at[0, pl.ds(sid * SEG_T, SEG_T)], a_v)

    def comb(src_t, carry):
        pltpu.sync_copy(stage.at[src_t, pl.ds(sid * SEG_T, SEG_T)], t_v)

        def vmax(i, carry2):
            sl = pl.ds(i * L, L)
            a_v[sl] = jnp.maximum(a_v[sl], t_v[sl])
            return carry2

        lax.fori_loop(0, SEG_T // L, vmax, 0)
        return carry

    lax.fori_loop(1, NSUB, comb, 0)
    pltpu.sync_copy(a_v, m_parts.at[cid, pl.ds(sid * SEG_T, SEG_T)])


def _s4_body(v1, v2, s1, s2, src1, dst1, src2, dst2, m_parts,
             den_parts, agg_parts,
             m_v, t_v, den_v, idx_s, idx_d, s_v, ex_v, vr,
             agg_sp, sem):
    cid = lax.axis_index("c")
    sid = lax.axis_index("s")
    wid = sid * NCORES + cid

    pltpu.sync_copy(m_parts.at[0], m_v)
    pltpu.sync_copy(m_parts.at[1], t_v)

    def mmax(i, carry):
        sl = pl.ds(i * L, L)
        m_v[sl] = jnp.maximum(m_v[sl], t_v[sl])
        den_v[sl] = jnp.zeros((L,), _f32)
        return carry

    lax.fori_loop(0, NSEG // L, mmax, 0)

    def zrow(r, carry):
        for j in range(8):
            vr[r, pl.ds(j * L, L)] = jnp.zeros((L,), _f32)
        return carry

    lax.fori_loop(0, CH, zrow, 0)
    for t in range(SEG_T // CH):
        pltpu.sync_copy(vr, agg_sp.at[pl.ds(sid * SEG_T + t * CH, CH), :])
    plsc.subcore_barrier()

    def do(vtab, scores, src, dst):
        def chunk(c, carry):
            base = (wid * CPW + c) * CH
            pltpu.sync_copy(src.at[pl.ds(base, CH)], idx_s)
            pltpu.sync_copy(dst.at[pl.ds(base, CH)], idx_d)
            pltpu.sync_copy(scores.at[pl.ds(base, CH)], s_v)
            cpv = pltpu.async_copy(vtab.at[idx_s], vr, sem)

            def grp(g, carry2):
                sl = pl.ds(g * L, L)
                dstv = idx_d[sl]
                mg = plsc.load_gather(m_v, [dstv])
                exv = jnp.exp(s_v[sl] - mg)
                ex_v[sl] = exv
                plsc.addupdate_scatter(den_v, [dstv], exv)
                return carry2

            lax.fori_loop(0, CH // L, grp, 0)
            cpv.wait()

            def edge(r, carry2):
                ev = plsc.load_gather(ex_v, [jnp.full((L,), r, _i32)])
                for j in range(8):
                    sl = pl.ds(j * L, L)
                    vr[r, sl] = vr[r, sl] * ev
                return carry2

            lax.fori_loop(0, CH, edge, 0)
            pltpu.sync_copy(vr, agg_sp.at[idx_d], add=True)
            return carry

        lax.fori_loop(0, CPW, chunk, 0)

    do(v1, s1, src1, dst1)
    do(v2, s2, src2, dst2)

    plsc.subcore_barrier()
    pltpu.sync_copy(den_v, den_parts.at[wid])
    for t in range(SEG_T // CH):
        sl = pl.ds(sid * SEG_T + t * CH, CH)
        pltpu.sync_copy(agg_sp.at[sl, :], vr)
        pltpu.sync_copy(vr, agg_parts.at[cid, sl, :])


def _pad_i32(x, n, val):
    x = x.astype(_i32)
    return jnp.pad(x, (0, n - x.shape[0]), constant_values=val)


def kernel(ft_user, ft_item, bn_g_u, bn_b_u, bn_g_i, bn_b_i,
           Wq_ui, bq_ui, Wk_ui, Wv_ui, attn_ui, emb_cnt,
           Wq_ii, bq_ii, Wk_ii, Wv_ii, attn_ii,
           W_agg, b_agg, W_self,
           src_ui, dst_ui, src_ii, dst_ii, cnt_ui):
    mesh = _mesh()

    r1 = lambda v: v.reshape(1, D)
    tc1 = pl.pallas_call(
        _tc1_body,
        out_shape=[jax.ShapeDtypeStruct((N_U, D), _f32)] * 7,
    )
    q1, k1, v1, q2, k2, v2, fti_n = tc1(
        ft_user, ft_item, r1(bn_g_u), r1(bn_b_u), r1(bn_g_i), r1(bn_b_i),
        Wq_ui, r1(bq_ui), Wk_ui, Wv_ui, Wq_ii, r1(bq_ii), Wk_ii, Wv_ii)

    src1 = _pad_i32(src_ui, EP, 0)
    dstg1 = _pad_i32(dst_ui, EP, 0)
    dsts1 = _pad_i32(dst_ui, EP, NSEG - 1)
    cnt1 = _pad_i32(cnt_ui, EP, 0)
    src2 = _pad_i32(src_ii, EP, 0)
    dstg2 = _pad_i32(dst_ii, EP, 0)
    dsts2 = _pad_i32(dst_ii, EP, NSEG - 1)

    s1_call = pl.kernel(
        _s1_body,
        out_type=[jax.ShapeDtypeStruct((EP,), _f32)] * 2,
        mesh=mesh,
        scratch_types=[
            pltpu.VMEM((CH,), _i32), pltpu.VMEM((CH,), _i32),
            pltpu.VMEM((CH,), _i32),
            pltpu.VMEM((CH, D), _f32), pltpu.VMEM((CH, D), _f32),
            pltpu.VMEM((CH, D), _f32),
            pltpu.VMEM((D,), _f32), pltpu.VMEM((CH,), _f32),
            pltpu.SemaphoreType.DMA, pltpu.SemaphoreType.DMA,
            pltpu.SemaphoreType.DMA,
        ],
        compiler_params=pltpu.CompilerParams(needs_layout_passes=False),
    )
    sc1, sc2 = s1_call(q1, k1, emb_cnt, attn_ui, q2, k2, attn_ii,
                       src1, dstg1, cnt1, src2, dstg2)

    s2_call = pl.kernel(
        _s2_body,
        out_type=jax.ShapeDtypeStruct((NCORES, NSEG), _f32),
        mesh=mesh,
        scratch_types=[
            pltpu.VMEM((NSEG,), _f32),
            pltpu.VMEM((CH,), _i32), pltpu.VMEM((CH,), _f32),
            pltpu.VMEM((SEG_T,), _f32), pltpu.VMEM((SEG_T,), _f32),
            pltpu.VMEM_SHARED((NSUB, NSEG), _f32),
        ],
        compiler_params=pltpu.CompilerParams(needs_layout_passes=False),
    )
    m_parts = s2_call(sc1, sc2, dsts1, dsts2)

    s4_call = pl.kernel(
        _s4_body,
        out_type=[
            jax.ShapeDtypeStruct((NW, NSEG), _f32),
            jax.ShapeDtypeStruct((NCORES, NSEG, D), _f32),
        ],
        mesh=mesh,
        scratch_types=[
            pltpu.VMEM((NSEG,), _f32), pltpu.VMEM((NSEG,), _f32),
            pltpu.VMEM((NSEG,), _f32),
            pltpu.VMEM((CH,), _i32), pltpu.VMEM((CH,), _i32),
            pltpu.VMEM((CH,), _f32), pltpu.VMEM((CH,), _f32),
            pltpu.VMEM((CH, D), _f32),
            pltpu.VMEM_SHARED((NSEG, D), _f32),
            pltpu.SemaphoreType.DMA,
        ],
        compiler_params=pltpu.CompilerParams(needs_layout_passes=False),
    )
    den_parts, agg_parts = s4_call(v1, v2, sc1, sc2,
                                   src1, dsts1, src2, dsts2, m_parts)

    tc2 = pl.pallas_call(
        _tc2_body,
        out_shape=jax.ShapeDtypeStruct((N_I, D), _f32),
    )
    return tc2(agg_parts, den_parts, fti_n, W_agg, r1(b_agg), W_self)


# speedup vs baseline: 5.9863x; 1.4566x over previous
import functools

import jax
import jax.numpy as jnp
from jax import lax
from jax.experimental import pallas as pl
from jax.experimental.pallas import tpu as pltpu
from jax.experimental.pallas import tpu_sc as plsc

N_U = 10000
N_I = 10000
E1 = 160000
E2 = 160000
D = 128

L = 16
NCORES = 2
NSUB = 16
NW = NCORES * NSUB
CH = 128
CPW = 40
EP = NW * CPW * CH
NSEG = 10240
SEG_T = NSEG // NSUB
NEG = -1e30

_f32 = jnp.float32
_i32 = jnp.int32


def _mesh():
    return plsc.VectorSubcoreMesh(
        core_axis_name="c", subcore_axis_name="s",
        num_cores=NCORES, num_subcores=NSUB)


_SC_PARAMS = None


def _tc1_body(ftu, fti, gu, bu, gi, bi,
              wq1, bq1, wk1, wv1, wq2, bq2, wk2, wv2, embi,
              q1o, k1o, v1o, q2o, k2o, v2o, ftio, embo):
    xu = ftu[...]
    mu = jnp.mean(xu, axis=0, keepdims=True)
    vu = jnp.mean((xu - mu) ** 2, axis=0, keepdims=True)
    xu = (xu - mu) / jnp.sqrt(vu + 1e-5) * gu[...] + bu[...]
    xi = fti[...]
    mi = jnp.mean(xi, axis=0, keepdims=True)
    vi = jnp.mean((xi - mi) ** 2, axis=0, keepdims=True)
    xi = (xi - mi) / jnp.sqrt(vi + 1e-5) * gi[...] + bi[...]
    ftio[...] = xi
    dot = functools.partial(jnp.dot, preferred_element_type=_f32)
    pad = jnp.zeros((NSEG - N_I, D), _f32)
    q1o[...] = -(dot(xu, wq1[...]) + bq1[...])
    k1o[...] = jnp.concatenate([-dot(xi, wk1[...]), pad], axis=0)
    v1o[...] = dot(xu, wv1[...])
    q2o[...] = -(dot(xi, wq2[...]) + bq2[...])
    k2o[...] = jnp.concatenate([-dot(xi, wk2[...]), pad], axis=0)
    v2o[...] = dot(xi, wv2[...])
    embo[...] = -embi[...]


def _tc2_body(aggp, denp, fti, wagg, bagg, wself, out):
    agg = aggp[0, :N_I, :] + aggp[1, :N_I, :]
    den = jnp.sum(denp[:, :N_I], axis=0)
    den = jnp.where(den > 0.0, den, 1.0)
    a = agg / den[:, None]
    dot = functools.partial(jnp.dot, preferred_element_type=_f32)
    out[...] = jnp.maximum(
        dot(a, wagg[...]) + dot(fti[...], wself[...]) + bagg[...], 0.0)


def _seg_max_update(m_priv, idxv, sv):
    def cond(st):
        cur = plsc.load_gather(m_priv, [idxv])
        return jnp.logical_and(st < L, jnp.any(cur < sv))

    def body(st):
        cur = plsc.load_gather(m_priv, [idxv])
        msk = cur < sv
        plsc.store_scatter(m_priv, [idxv], jnp.maximum(cur, sv), mask=msk)
        return st + 1

    lax.while_loop(cond, body, 0)


def _s1_body(q1, k1, emb, attn1, q2, k2, attn2,
             src1, dst1, cnt1, src2, dst2,
             s1_out, s2_out, m_parts,
             ia0, ia1, ib0, ib1, ic0, ic1,
             qr0, qr1, kr0, kr1, cr0, cr1,
             attn_v, sc_v, m_priv, a_v, t_v, stage,
             sia0, sia1, sib0, sib1, sic0, sic1,
             sq0, sq1, sk0, sk1, scn0, scn1):
    cid = lax.axis_index("c")
    sid = lax.axis_index("s")
    wid = sid * NCORES + cid
    ia = (ia0, ia1)
    ib = (ib0, ib1)
    ic = (ic0, ic1)
    qr = (qr0, qr1)
    kr = (kr0, kr1)
    cr = (cr0, cr1)
    sia = (sia0, sia1)
    sib = (sib0, sib1)
    sic = (sic0, sic1)
    sq = (sq0, sq1)
    sk = (sk0, sk1)
    scn = (scn0, scn1)

    def ini(i, carry):
        m_priv[pl.ds(i * L, L)] = jnp.full((L,), NEG, _f32)
        return carry

    lax.fori_loop(0, NSEG // L, ini, 0)

    def do_etype(qtab, ktab, attn_hbm, src, dst, cnt, out, has_cnt):
        pltpu.sync_copy(attn_hbm, attn_v)

        def fire_idx(c, b):
            base = (wid * CPW + c) * CH
            pltpu.async_copy(src.at[pl.ds(base, CH)], ia[b], sia[b])
            pltpu.async_copy(dst.at[pl.ds(base, CH)], ib[b], sib[b])
            if has_cnt:
                pltpu.async_copy(cnt.at[pl.ds(base, CH)], ic[b], sic[b])

        def wait_idx(b):
            pltpu.make_async_copy(src.at[pl.ds(0, CH)], ia[b], sia[b]).wait()
            pltpu.make_async_copy(dst.at[pl.ds(0, CH)], ib[b], sib[b]).wait()
            if has_cnt:
                pltpu.make_async_copy(cnt.at[pl.ds(0, CH)], ic[b],
                                      sic[b]).wait()

        def fire_rows(b):
            pltpu.async_copy(qtab.at[ia[b]], qr[b], sq[b])
            pltpu.async_copy(ktab.at[ib[b]], kr[b], sk[b])
            if has_cnt:
                pltpu.async_copy(emb.at[ic[b]], cr[b], scn[b])

        def wait_rows(b):
            pltpu.make_async_copy(qtab.at[ia[b]], qr[b], sq[b]).wait()
            pltpu.make_async_copy(ktab.at[ib[b]], kr[b], sk[b]).wait()
            if has_cnt:
                pltpu.make_async_copy(emb.at[ic[b]], cr[b], scn[b]).wait()

        def compute(c, b):
            base = (wid * CPW + c) * CH
            qrb, krb, crb = qr[b], kr[b], cr[b]

            def grp(g, carry2):
                def edge(r2, vec):
                    r = g * L + r2
                    acc = jnp.zeros((L,), _f32)
                    for j in range(8):
                        sl = pl.ds(j * L, L)
                        tn = qrb[r, sl] + krb[r, sl]
                        if has_cnt:
                            tn = tn + crb[r, sl]
                        acc = acc + attn_v[sl] / (1.0 + jnp.exp(tn))
                    s = jnp.sum(acc)
                    lane = lax.broadcasted_iota(_i32, (L,), 0)
                    return jnp.where(lane == r2, s, vec)

                vec = lax.fori_loop(0, L, edge, jnp.zeros((L,), _f32))
                sc_v[pl.ds(g * L, L)] = vec
                dstv = ib[b][pl.ds(g * L, L)]
                _seg_max_update(m_priv, dstv, vec)
                return carry2

            lax.fori_loop(0, CH // L, grp, 0)
            pltpu.sync_copy(sc_v, out.at[pl.ds(base, CH)])

        fire_idx(0, 0)
        wait_idx(0)
        fire_rows(0)
        fire_idx(1, 1)

        def step(p, carry):
            for b in (0, 1):
                c = 2 * p + b

                @pl.when(c + 1 < CPW)
                def _(b=b):
                    wait_idx(1 - b)
                    fire_rows(1 - b)

                wait_rows(b)
                compute(c, b)

                @pl.when(c + 2 < CPW)
                def _(b=b, c=c):
                    fire_idx(c + 2, b)

            return carry

        lax.fori_loop(0, CPW // 2, step, 0)

    do_etype(q1, k1, attn1, src1, dst1, cnt1, s1_out, True)
    do_etype(q2, k2, attn2, src2, dst2, None, s2_out, False)

    pltpu.sync_copy(m_priv, stage.at[sid])
    plsc.subcore_barrier()
    pltpu.sync_copy(stage.at[0, pl.ds(sid * SEG_T, SEG_T)], a_v)

    def comb(src_t, carry):
        pltpu.sync_copy(stage.at[src_t, pl.ds(sid * SEG_T, SEG_T)], t_v)

        def vmax(i, carry2):
            sl = pl.ds(i * L, L)
            a_v[sl] = jnp.maximum(a_v[sl], t_v[sl])
            return carry2

        lax.fori_loop(0, SEG_T // L, vmax, 0)
        return carry

    lax.fori_loop(1, NSUB, comb, 0)
    pltpu.sync_copy(a_v, m_parts.at[cid, pl.ds(sid * SEG_T, SEG_T)])


def _s4_body(v1, v2, s1, s2, src1, dst1, src2, dst2, m_parts,
             den_parts, agg_parts,
             m_v,
             is0, is1, id0, id1, sv0, sv1, ex_v,
             vr0, vr1,
             den_sp, agg_sp,
             sis0, sis1, sid_0, sid_1, ssv0, ssv1, svr0, svr1):
    cid = lax.axis_index("c")
    sid = lax.axis_index("s")
    wid = sid * NCORES + cid
    isb = (is0, is1)
    idb = (id0, id1)
    svb = (sv0, sv1)
    vrb = (vr0, vr1)
    sis = (sis0, sis1)
    sdd = (sid_0, sid_1)
    ssv = (ssv0, ssv1)
    svr = (svr0, svr1)

    pltpu.sync_copy(m_parts.at[0], m_v)

    def mchunk(p, carry):
        pltpu.sync_copy(m_parts.at[1, pl.ds(p * CH, CH)], sv0)

        def mmax(i, carry2):
            sl = pl.ds(i * L, L)
            gsl = pl.ds(p * CH + i * L, L)
            m_v[gsl] = jnp.maximum(m_v[gsl], sv0[sl])
            return carry2

        lax.fori_loop(0, CH // L, mmax, 0)
        return carry

    lax.fori_loop(0, NSEG // CH, mchunk, 0)

    def zrow(r, carry):
        for j in range(8):
            vr0[r, pl.ds(j * L, L)] = jnp.zeros((L,), _f32)
        return carry

    lax.fori_loop(0, CH, zrow, 0)

    def zex(i, carry):
        ex_v[pl.ds(i * L, L)] = jnp.zeros((L,), _f32)
        return carry

    lax.fori_loop(0, CH // L, zex, 0)
    for t in range(SEG_T // CH):
        pltpu.sync_copy(vr0, agg_sp.at[pl.ds(sid * SEG_T + t * CH, CH), :])
        pltpu.sync_copy(ex_v, den_sp.at[pl.ds(sid * SEG_T + t * CH, CH)])
    plsc.subcore_barrier()

    def do(vtab, scores, src, dst):
        def fire_idx(c, b):
            base = (wid * CPW + c) * CH
            pltpu.async_copy(src.at[pl.ds(base, CH)], isb[b], sis[b])
            pltpu.async_copy(dst.at[pl.ds(base, CH)], idb[b], sdd[b])
            pltpu.async_copy(scores.at[pl.ds(base, CH)], svb[b], ssv[b])

        def wait_idx(b):
            pltpu.make_async_copy(src.at[pl.ds(0, CH)], isb[b], sis[b]).wait()
            pltpu.make_async_copy(dst.at[pl.ds(0, CH)], idb[b], sdd[b]).wait()
            pltpu.make_async_copy(scores.at[pl.ds(0, CH)], svb[b],
                                  ssv[b]).wait()

        def fire_rows(b):
            pltpu.async_copy(vtab.at[isb[b]], vrb[b], svr[b])

        def wait_rows(b):
            pltpu.make_async_copy(vtab.at[isb[b]], vrb[b], svr[b]).wait()

        def compute(c, b):
            vrc = vrb[b]

            def grp(g, carry2):
                sl = pl.ds(g * L, L)
                dstv = idb[b][sl]
                mg = plsc.load_gather(m_v, [dstv])
                exv = jnp.exp(svb[b][sl] - mg)
                ex_v[sl] = exv
                return carry2

            lax.fori_loop(0, CH // L, grp, 0)
            pltpu.sync_copy(ex_v, den_sp.at[idb[b]], add=True)
            wait_rows(b)

            def edge(r, carry2):
                ev = plsc.load_gather(ex_v, [jnp.full((L,), r, _i32)])
                for j in range(8):
                    sl = pl.ds(j * L, L)
                    vrc[r, sl] = vrc[r, sl] * ev
                return carry2

            lax.fori_loop(0, CH, edge, 0)
            pltpu.sync_copy(vrc, agg_sp.at[idb[b]], add=True)

        fire_idx(0, 0)
        wait_idx(0)
        fire_rows(0)
        fire_idx(1, 1)

        def step(p, carry):
            for b in (0, 1):
                c = 2 * p + b

                @pl.when(c + 1 < CPW)
                def _(b=b):
                    wait_idx(1 - b)
                    fire_rows(1 - b)

                compute(c, b)

                @pl.when(c + 2 < CPW)
                def _(b=b, c=c):
                    fire_idx(c + 2, b)

            return carry

        lax.fori_loop(0, CPW // 2, step, 0)

    do(v1, s1, src1, dst1)
    do(v2, s2, src2, dst2)

    plsc.subcore_barrier()
    for t in range(SEG_T // CH):
        sl = pl.ds(sid * SEG_T + t * CH, CH)
        pltpu.sync_copy(agg_sp.at[sl, :], vr0)
        pltpu.sync_copy(vr0, agg_parts.at[cid, sl, :])
        pltpu.sync_copy(den_sp.at[sl], ex_v)
        pltpu.sync_copy(ex_v, den_parts.at[cid, sl])


def _pad_i32(x, n, val):
    x = x.astype(_i32)
    return jnp.pad(x, (0, n - x.shape[0]), constant_values=val)


def kernel(ft_user, ft_item, bn_g_u, bn_b_u, bn_g_i, bn_b_i,
           Wq_ui, bq_ui, Wk_ui, Wv_ui, attn_ui, emb_cnt,
           Wq_ii, bq_ii, Wk_ii, Wv_ii, attn_ii,
           W_agg, b_agg, W_self,
           src_ui, dst_ui, src_ii, dst_ii, cnt_ui):
    mesh = _mesh()
    scp = pltpu.CompilerParams(needs_layout_passes=False)

    r1 = lambda v: v.reshape(1, D)
    tc1 = pl.pallas_call(
        _tc1_body,
        out_shape=[
            jax.ShapeDtypeStruct((N_U, D), _f32),
            jax.ShapeDtypeStruct((NSEG, D), _f32),
            jax.ShapeDtypeStruct((N_U, D), _f32),
            jax.ShapeDtypeStruct((N_U, D), _f32),
            jax.ShapeDtypeStruct((NSEG, D), _f32),
            jax.ShapeDtypeStruct((N_U, D), _f32),
            jax.ShapeDtypeStruct((N_U, D), _f32),
            jax.ShapeDtypeStruct((100, D), _f32),
        ],
    )
    q1, k1, v1, q2, k2, v2, fti_n, emb_n = tc1(
        ft_user, ft_item, r1(bn_g_u), r1(bn_b_u), r1(bn_g_i), r1(bn_b_i),
        Wq_ui, r1(bq_ui), Wk_ui, Wv_ui, Wq_ii, r1(bq_ii), Wk_ii, Wv_ii,
        emb_cnt)

    src1 = _pad_i32(src_ui, EP, 0)
    dst1 = _pad_i32(dst_ui, EP, NSEG - 1)
    cnt1 = _pad_i32(cnt_ui, EP, 0)
    src2 = _pad_i32(src_ii, EP, 0)
    dst2 = _pad_i32(dst_ii, EP, NSEG - 1)

    s1_call = pl.kernel(
        _s1_body,
        out_type=[
            jax.ShapeDtypeStruct((EP,), _f32),
            jax.ShapeDtypeStruct((EP,), _f32),
            jax.ShapeDtypeStruct((NCORES, NSEG), _f32),
        ],
        mesh=mesh,
        scratch_types=[
            pltpu.VMEM((CH,), _i32), pltpu.VMEM((CH,), _i32),
            pltpu.VMEM((CH,), _i32), pltpu.VMEM((CH,), _i32),
            pltpu.VMEM((CH,), _i32), pltpu.VMEM((CH,), _i32),
            pltpu.VMEM((CH, D), _f32), pltpu.VMEM((CH, D), _f32),
            pltpu.VMEM((CH, D), _f32), pltpu.VMEM((CH, D), _f32),
            pltpu.VMEM((CH, D), _f32), pltpu.VMEM((CH, D), _f32),
            pltpu.VMEM((D,), _f32), pltpu.VMEM((CH,), _f32),
            pltpu.VMEM((NSEG,), _f32),
            pltpu.VMEM((SEG_T,), _f32), pltpu.VMEM((SEG_T,), _f32),
            pltpu.VMEM_SHARED((NSUB, NSEG), _f32),
        ] + [pltpu.SemaphoreType.DMA] * 12,
        compiler_params=scp,
    )
    sc1, sc2, m_parts = s1_call(q1, k1, emb_n, attn_ui, q2, k2, attn_ii,
                                src1, dst1, cnt1, src2, dst2)

    s4_call = pl.kernel(
        _s4_body,
        out_type=[
            jax.ShapeDtypeStruct((NCORES, NSEG), _f32),
            jax.ShapeDtypeStruct((NCORES, NSEG, D), _f32),
        ],
        mesh=mesh,
        scratch_types=[
            pltpu.VMEM((NSEG,), _f32),
            pltpu.VMEM((CH,), _i32), pltpu.VMEM((CH,), _i32),
            pltpu.VMEM((CH,), _i32), pltpu.VMEM((CH,), _i32),
            pltpu.VMEM((CH,), _f32), pltpu.VMEM((CH,), _f32),
            pltpu.VMEM((CH,), _f32),
            pltpu.VMEM((CH, D), _f32), pltpu.VMEM((CH, D), _f32),
            pltpu.VMEM_SHARED((NSEG,), _f32),
            pltpu.VMEM_SHARED((NSEG, D), _f32),
        ] + [pltpu.SemaphoreType.DMA] * 8,
        compiler_params=scp,
    )
    den_parts, agg_parts = s4_call(v1, v2, sc1, sc2,
                                   src1, dst1, src2, dst2, m_parts)

    tc2 = pl.pallas_call(
        _tc2_body,
        out_shape=jax.ShapeDtypeStruct((N_I, D), _f32),
    )
    return tc2(agg_parts, den_parts, fti_n, W_agg, r1(b_agg), W_self)
